# baseline (speedup 1.0000x reference)
import functools

import jax
import jax.numpy as jnp
from jax import lax
from jax.experimental import pallas as pl
from jax.experimental.pallas import tpu as pltpu
from jax.experimental.pallas import tpu_sc as plsc

VOCAB = 100000
EMB = 128
HID = 1024
NCLS = 2
B = 16384
L = 20

NC = 2
NS = 16
NW = NC * NS
CB = 16
IDS_PER_CHUNK = CB * L
IDX_COLS = 64
STREAMS_PER_CHUNK = IDS_PER_CHUNK // IDX_COLS

_sc_mesh = plsc.VectorSubcoreMesh(core_axis_name="c", subcore_axis_name="s")


def _make_gather_sum(nb):
    b_per_w = nb // NW
    nch = b_per_w // CB
    idx_rows = b_per_w * L // IDX_COLS

    @functools.partial(
        pl.kernel,
        mesh=_sc_mesh,
        out_type=jax.ShapeDtypeStruct((nb, EMB), jnp.float32),
        scratch_types=[
            pltpu.VMEM((idx_rows, IDX_COLS), jnp.int32),
            pltpu.VMEM((2, IDS_PER_CHUNK, EMB), jnp.float32),
            pltpu.VMEM((CB, EMB), jnp.float32),
            pltpu.SemaphoreType.DMA,
            pltpu.SemaphoreType.DMA,
        ],
    )
    def gather_sum(ids_hbm, table_hbm, out_hbm, idx_v, rows_v, out_v, sem0, sem1):
        wid = lax.axis_index("s") * NC + lax.axis_index("c")
        base_row = wid * b_per_w
        sems = (sem0, sem1)

        pltpu.sync_copy(ids_hbm.at[pl.ds(wid * idx_rows, idx_rows)], idx_v)

        def fire(ch, slot):
            for k in range(STREAMS_PER_CHUNK):
                pltpu.async_copy(
                    table_hbm.at[idx_v.at[ch * STREAMS_PER_CHUNK + k]],
                    rows_v.at[slot, pl.ds(IDX_COLS * k, IDX_COLS)],
                    sems[slot],
                )

        def drain(slot):
            for k in range(STREAMS_PER_CHUNK):
                pltpu.make_async_copy(
                    table_hbm.at[idx_v.at[k]],
                    rows_v.at[slot, pl.ds(IDX_COLS * k, IDX_COLS)],
                    sems[slot],
                ).wait()

        def reduce_store(ch, slot):
            def row_body(i, c2):
                rbase = i * L
                acc = [rows_v[slot, rbase, pl.ds(16 * d, 16)] for d in range(EMB // 16)]
                for l in range(1, L):
                    for d in range(EMB // 16):
                        acc[d] = acc[d] + rows_v[slot, rbase + l, pl.ds(16 * d, 16)]
                for d in range(EMB // 16):
                    out_v[i, pl.ds(16 * d, 16)] = acc[d]
                return c2

            lax.fori_loop(0, CB, row_body, 0)
            pltpu.sync_copy(out_v, out_hbm.at[pl.ds(base_row + ch * CB, CB)])

        fire(0, 0)

        def body2(j, carry):
            ch0 = 2 * j
            fire(ch0 + 1, 1)
            drain(0)
            reduce_store(ch0, 0)

            @pl.when(ch0 + 2 < nch)
            def _():
                fire(ch0 + 2, 0)

            drain(1)
            reduce_store(ch0 + 1, 1)
            return carry

        lax.fori_loop(0, nch // 2, body2, 0)

    return gather_sum


NHALF = 4
_gather_sum_half = _make_gather_sum(B // NHALF)


def _mlp_body(x_ref, w1_ref, b1_ref, w2_ref, b2_ref, o_ref):
    h = jnp.dot(x_ref[...], w1_ref[...], preferred_element_type=jnp.float32)
    h = jnp.maximum(h + b1_ref[...], 0.0)
    o_ref[...] = (
        jnp.dot(h, w2_ref[...], preferred_element_type=jnp.float32) + b2_ref[...]
    )


def _mlp(x, W1, b1, W2, b2):
    BM = 1024
    nb = x.shape[0]
    grid = (nb // BM,)
    return pl.pallas_call(
        _mlp_body,
        grid=grid,
        in_specs=[
            pl.BlockSpec((BM, EMB), lambda i: (i, 0)),
            pl.BlockSpec((EMB, HID), lambda i: (0, 0)),
            pl.BlockSpec((1, HID), lambda i: (0, 0)),
            pl.BlockSpec((HID, NCLS), lambda i: (0, 0)),
            pl.BlockSpec((1, NCLS), lambda i: (0, 0)),
        ],
        out_specs=pl.BlockSpec((BM, NCLS), lambda i: (i, 0)),
        out_shape=jax.ShapeDtypeStruct((nb, NCLS), jnp.float32),
    )(x, W1, b1, W2, b2)


def kernel(batch_inputs, batch_lengths, table, W1, b1, W2, b2):
    del batch_lengths
    ids = batch_inputs.reshape(B * L // IDX_COLS, IDX_COLS)
    rows_per_half = ids.shape[0] // NHALF
    W1s = W1 * jnp.float32(1.0 / L)
    b1r = b1.reshape(1, HID)
    b2r = b2.reshape(1, NCLS)
    outs = []
    for h in range(NHALF):
        ids_h = lax.slice_in_dim(ids, h * rows_per_half, (h + 1) * rows_per_half)
        x_h = _gather_sum_half(ids_h, table)
        outs.append(_mlp(x_h, W1s, b1r, W2, b2r))
    return jnp.concatenate(outs, axis=0)


# speedup vs baseline: 1.0903x; 1.0903x over previous
import functools

import jax
import jax.numpy as jnp
from jax import lax
from jax.experimental import pallas as pl
from jax.experimental.pallas import tpu as pltpu
from jax.experimental.pallas import tpu_sc as plsc

VOCAB = 100000
EMB = 128
HID = 1024
NCLS = 2
B = 16384
L = 20

NC = 2
NS = 16
NW = NC * NS
CB = 16
IDS_PER_CHUNK = CB * L
IDX_COLS = 64
STREAMS_PER_CHUNK = IDS_PER_CHUNK // IDX_COLS

_sc_mesh = plsc.VectorSubcoreMesh(core_axis_name="c", subcore_axis_name="s")


def _make_gather_sum(nb):
    b_per_w = nb // NW
    nch = b_per_w // CB
    idx_rows = b_per_w * L // IDX_COLS

    @functools.partial(
        pl.kernel,
        mesh=_sc_mesh,
        out_type=jax.ShapeDtypeStruct((nb, EMB), jnp.float32),
        scratch_types=[
            pltpu.VMEM((idx_rows, IDX_COLS), jnp.int32),
            pltpu.VMEM((2, IDS_PER_CHUNK, EMB), jnp.float32),
            pltpu.VMEM((CB, EMB), jnp.float32),
            pltpu.SemaphoreType.DMA,
            pltpu.SemaphoreType.DMA,
        ],
    )
    def gather_sum(ids_hbm, table_hbm, out_hbm, idx_v, rows_v, out_v, sem0, sem1):
        wid = lax.axis_index("s") * NC + lax.axis_index("c")
        base_row = wid * b_per_w
        sems = (sem0, sem1)

        pltpu.sync_copy(ids_hbm.at[pl.ds(wid * idx_rows, idx_rows)], idx_v)

        scale = jnp.float32(1.0 / L)

        def fire(ch, slot):
            for k in range(STREAMS_PER_CHUNK):
                pltpu.async_copy(
                    table_hbm.at[idx_v.at[ch * STREAMS_PER_CHUNK + k]],
                    rows_v.at[slot, pl.ds(IDX_COLS * k, IDX_COLS)],
                    sems[slot],
                )

        def drain(slot):
            for k in range(STREAMS_PER_CHUNK):
                pltpu.make_async_copy(
                    table_hbm.at[idx_v.at[k]],
                    rows_v.at[slot, pl.ds(IDX_COLS * k, IDX_COLS)],
                    sems[slot],
                ).wait()

        def reduce_store(ch, slot):
            def row_body(i, c2):
                rbase = i * L
                acc = [rows_v[slot, rbase, pl.ds(16 * d, 16)] for d in range(EMB // 16)]
                for l in range(1, L):
                    for d in range(EMB // 16):
                        acc[d] = acc[d] + rows_v[slot, rbase + l, pl.ds(16 * d, 16)]
                for d in range(EMB // 16):
                    out_v[i, pl.ds(16 * d, 16)] = acc[d] * scale
                return c2

            lax.fori_loop(0, CB, row_body, 0)
            pltpu.sync_copy(out_v, out_hbm.at[pl.ds(base_row + ch * CB, CB)])

        fire(0, 0)

        def body2(j, carry):
            ch0 = 2 * j
            fire(ch0 + 1, 1)
            drain(0)
            reduce_store(ch0, 0)

            @pl.when(ch0 + 2 < nch)
            def _():
                fire(ch0 + 2, 0)

            drain(1)
            reduce_store(ch0 + 1, 1)
            return carry

        lax.fori_loop(0, nch // 2, body2, 0)

    return gather_sum


NHALF = 2
_gather_sum_half = _make_gather_sum(B // NHALF)


def _mlp_body(x_ref, w1_ref, b1_ref, w2_ref, b2_ref, o_ref):
    h = jnp.dot(x_ref[...], w1_ref[...], preferred_element_type=jnp.float32)
    h = jnp.maximum(h + b1_ref[...], 0.0)
    o_ref[...] = (
        jnp.dot(h, w2_ref[...], preferred_element_type=jnp.float32) + b2_ref[...]
    )


def _mlp(x, W1, b1, W2, b2):
    BM = 1024
    nb = x.shape[0]
    grid = (nb // BM,)
    return pl.pallas_call(
        _mlp_body,
        grid=grid,
        in_specs=[
            pl.BlockSpec((BM, EMB), lambda i: (i, 0)),
            pl.BlockSpec((EMB, HID), lambda i: (0, 0)),
            pl.BlockSpec((1, HID), lambda i: (0, 0)),
            pl.BlockSpec((HID, NCLS), lambda i: (0, 0)),
            pl.BlockSpec((1, NCLS), lambda i: (0, 0)),
        ],
        out_specs=pl.BlockSpec((BM, NCLS), lambda i: (i, 0)),
        out_shape=jax.ShapeDtypeStruct((nb, NCLS), jnp.float32),
    )(x, W1, b1, W2, b2)


def kernel(batch_inputs, batch_lengths, table, W1, b1, W2, b2):
    del batch_lengths
    ids = batch_inputs.reshape(B * L // IDX_COLS, IDX_COLS)
    rows_per_half = ids.shape[0] // NHALF
    b1r = b1.reshape(1, HID)
    b2r = b2.reshape(1, NCLS)
    outs = []
    for h in range(NHALF):
        ids_h = lax.slice_in_dim(ids, h * rows_per_half, (h + 1) * rows_per_half)
        x_h = _gather_sum_half(ids_h, table)
        outs.append(_mlp(x_h, W1, b1r, W2, b2r))
    return jnp.concatenate(outs, axis=0)


# speedup vs baseline: 1.1184x; 1.0258x over previous
import functools

import jax
import jax.numpy as jnp
from jax import lax
from jax.experimental import pallas as pl
from jax.experimental.pallas import tpu as pltpu
from jax.experimental.pallas import tpu_sc as plsc

VOCAB = 100000
EMB = 128
HID = 1024
NCLS = 2
B = 16384
L = 20

NC = 2
NS = 16
NW = NC * NS
CB = 16
IDS_PER_CHUNK = CB * L
IDX_COLS = 64
STREAMS_PER_CHUNK = IDS_PER_CHUNK // IDX_COLS

_sc_mesh = plsc.VectorSubcoreMesh(core_axis_name="c", subcore_axis_name="s")


def _make_gather_sum(nb):
    b_per_w = nb // NW
    nch = b_per_w // CB
    idx_rows = b_per_w * L // IDX_COLS

    @functools.partial(
        pl.kernel,
        mesh=_sc_mesh,
        out_type=jax.ShapeDtypeStruct((nb, EMB), jnp.float32),
        scratch_types=[
            pltpu.VMEM((idx_rows, IDX_COLS), jnp.int32),
            pltpu.VMEM((2, IDS_PER_CHUNK, EMB), jnp.float32),
            pltpu.VMEM((CB, EMB), jnp.float32),
            pltpu.SemaphoreType.DMA,
            pltpu.SemaphoreType.DMA,
        ],
    )
    def gather_sum(ids_hbm, table_hbm, out_hbm, idx_v, rows_v, out_v, sem0, sem1):
        wid = lax.axis_index("s") * NC + lax.axis_index("c")
        base_row = wid * b_per_w
        sems = (sem0, sem1)

        pltpu.sync_copy(ids_hbm.at[pl.ds(wid * idx_rows, idx_rows)], idx_v)

        scale = jnp.float32(1.0 / L)

        def fire(ch, slot):
            for k in range(STREAMS_PER_CHUNK):
                pltpu.async_copy(
                    table_hbm.at[idx_v.at[ch * STREAMS_PER_CHUNK + k]],
                    rows_v.at[slot, pl.ds(IDX_COLS * k, IDX_COLS)],
                    sems[slot],
                )

        def drain(slot):
            for k in range(STREAMS_PER_CHUNK):
                pltpu.make_async_copy(
                    table_hbm.at[idx_v.at[k]],
                    rows_v.at[slot, pl.ds(IDX_COLS * k, IDX_COLS)],
                    sems[slot],
                ).wait()

        def reduce_store(ch, slot):
            def row_body(i, c2):
                rbase = i * L
                acc = [rows_v[slot, rbase, pl.ds(16 * d, 16)] for d in range(EMB // 16)]
                for l in range(1, L):
                    for d in range(EMB // 16):
                        acc[d] = acc[d] + rows_v[slot, rbase + l, pl.ds(16 * d, 16)]
                for d in range(EMB // 16):
                    out_v[i, pl.ds(16 * d, 16)] = acc[d] * scale
                return c2

            lax.fori_loop(0, CB, row_body, 0)
            pltpu.sync_copy(out_v, out_hbm.at[pl.ds(base_row + ch * CB, CB)])

        fire(0, 0)

        def body2(j, carry):
            ch0 = 2 * j
            fire(ch0 + 1, 1)
            drain(0)
            reduce_store(ch0, 0)

            @pl.when(ch0 + 2 < nch)
            def _():
                fire(ch0 + 2, 0)

            drain(1)
            reduce_store(ch0 + 1, 1)
            return carry

        lax.fori_loop(0, nch // 2, body2, 0)

    return gather_sum


NHALF = 2
_gather_sum_half = _make_gather_sum(B // NHALF)


def _mlp_body(x_ref, w1_ref, b1_ref, w2_ref, b2_ref, prev_ref, o_ref):
    del prev_ref
    h = jnp.dot(x_ref[...], w1_ref[...], preferred_element_type=jnp.float32)
    h = jnp.maximum(h + b1_ref[...], 0.0)
    o_ref[...] = (
        jnp.dot(h, w2_ref[...], preferred_element_type=jnp.float32) + b2_ref[...]
    )


def _mlp_into(x, W1, b1, W2, b2, prev, off_blocks):
    BM = 1024
    nb = x.shape[0]
    grid = (nb // BM,)
    return pl.pallas_call(
        _mlp_body,
        grid=grid,
        in_specs=[
            pl.BlockSpec((BM, EMB), lambda i: (i, 0)),
            pl.BlockSpec((EMB, HID), lambda i: (0, 0)),
            pl.BlockSpec((1, HID), lambda i: (0, 0)),
            pl.BlockSpec((HID, NCLS), lambda i: (0, 0)),
            pl.BlockSpec((1, NCLS), lambda i: (0, 0)),
            pl.BlockSpec(memory_space=pl.ANY),
        ],
        out_specs=pl.BlockSpec((BM, NCLS), lambda i: (i + off_blocks, 0)),
        out_shape=jax.ShapeDtypeStruct((B, NCLS), jnp.float32),
        input_output_aliases={5: 0},
    )(x, W1, b1, W2, b2, prev)


def kernel(batch_inputs, batch_lengths, table, W1, b1, W2, b2):
    del batch_lengths
    ids = batch_inputs.reshape(B * L // IDX_COLS, IDX_COLS)
    rows_per_half = ids.shape[0] // NHALF
    b1r = b1.reshape(1, HID)
    b2r = b2.reshape(1, NCLS)
    out = jnp.zeros((B, NCLS), jnp.float32)
    for h in range(NHALF):
        ids_h = lax.slice_in_dim(ids, h * rows_per_half, (h + 1) * rows_per_half)
        x_h = _gather_sum_half(ids_h, table)
        out = _mlp_into(x_h, W1, b1r, W2, b2r, out,
                        h * (B // NHALF) // 1024)
    return out


# speedup vs baseline: 1.1285x; 1.0090x over previous
import functools

import jax
import jax.numpy as jnp
from jax import lax
from jax.experimental import pallas as pl
from jax.experimental.pallas import tpu as pltpu
from jax.experimental.pallas import tpu_sc as plsc

VOCAB = 100000
EMB = 128
HID = 1024
NCLS = 2
B = 16384
L = 20

NC = 2
NS = 16
NW = NC * NS
CB = 16
IDS_PER_CHUNK = CB * L
IDX_COLS = 64
STREAMS_PER_CHUNK = IDS_PER_CHUNK // IDX_COLS

_sc_mesh = plsc.VectorSubcoreMesh(core_axis_name="c", subcore_axis_name="s")


def _make_gather_sum(nb):
    b_per_w = nb // NW
    nch = b_per_w // CB
    idx_rows = b_per_w * L // IDX_COLS

    @functools.partial(
        pl.kernel,
        mesh=_sc_mesh,
        out_type=jax.ShapeDtypeStruct((nb, EMB), jnp.float32),
        scratch_types=[
            pltpu.VMEM((idx_rows, IDX_COLS), jnp.int32),
            pltpu.VMEM((2, IDS_PER_CHUNK, EMB), jnp.float32),
            pltpu.VMEM((CB, EMB), jnp.float32),
            pltpu.SemaphoreType.DMA,
            pltpu.SemaphoreType.DMA,
        ],
    )
    def gather_sum(ids_hbm, table_hbm, out_hbm, idx_v, rows_v, out_v, sem0, sem1):
        wid = lax.axis_index("s") * NC + lax.axis_index("c")
        base_row = wid * b_per_w
        sems = (sem0, sem1)

        pltpu.sync_copy(ids_hbm.at[pl.ds(wid * idx_rows, idx_rows)], idx_v)

        scale = jnp.float32(1.0 / L)

        def fire(ch, slot):
            for k in range(STREAMS_PER_CHUNK):
                pltpu.async_copy(
                    table_hbm.at[idx_v.at[ch * STREAMS_PER_CHUNK + k]],
                    rows_v.at[slot, pl.ds(IDX_COLS * k, IDX_COLS)],
                    sems[slot],
                )

        def drain(slot):
            for k in range(STREAMS_PER_CHUNK):
                pltpu.make_async_copy(
                    table_hbm.at[idx_v.at[k]],
                    rows_v.at[slot, pl.ds(IDX_COLS * k, IDX_COLS)],
                    sems[slot],
                ).wait()

        def reduce_store(ch, slot):
            def row_body(i, c2):
                rbase = i * L
                acc = [rows_v[slot, rbase, pl.ds(16 * d, 16)] for d in range(EMB // 16)]
                for l in range(1, L):
                    for d in range(EMB // 16):
                        acc[d] = acc[d] + rows_v[slot, rbase + l, pl.ds(16 * d, 16)]
                for d in range(EMB // 16):
                    out_v[i, pl.ds(16 * d, 16)] = acc[d] * scale
                return c2

            lax.fori_loop(0, CB, row_body, 0)
            pltpu.sync_copy(out_v, out_hbm.at[pl.ds(base_row + ch * CB, CB)])

        fire(0, 0)

        def body2(j, carry):
            ch0 = 2 * j
            fire(ch0 + 1, 1)
            drain(0)
            reduce_store(ch0, 0)

            @pl.when(ch0 + 2 < nch)
            def _():
                fire(ch0 + 2, 0)

            drain(1)
            reduce_store(ch0 + 1, 1)
            return carry

        lax.fori_loop(0, nch // 2, body2, 0)

    return gather_sum


NHALF = 2
_gather_sum_half = _make_gather_sum(B // NHALF)


def _mlp_body(x_ref, w1_ref, b1_ref, w2_ref, b2_ref, prev_ref, o_ref):
    del prev_ref
    h = jnp.dot(x_ref[...], w1_ref[...], preferred_element_type=jnp.float32)
    h = jnp.maximum(h + b1_ref[...], 0.0)
    o_ref[...] = (
        jnp.dot(h, w2_ref[...], preferred_element_type=jnp.float32) + b2_ref[...]
    )


def _mlp_into(x, W1, b1, W2, b2, prev, off_blocks):
    BM = 2048
    nb = x.shape[0]
    grid = (nb // BM,)
    return pl.pallas_call(
        _mlp_body,
        grid=grid,
        in_specs=[
            pl.BlockSpec((BM, EMB), lambda i: (i, 0)),
            pl.BlockSpec((EMB, HID), lambda i: (0, 0)),
            pl.BlockSpec((1, HID), lambda i: (0, 0)),
            pl.BlockSpec((HID, NCLS), lambda i: (0, 0)),
            pl.BlockSpec((1, NCLS), lambda i: (0, 0)),
            pl.BlockSpec(memory_space=pl.ANY),
        ],
        out_specs=pl.BlockSpec((BM, NCLS), lambda i: (i + off_blocks, 0)),
        out_shape=jax.ShapeDtypeStruct((B, NCLS), jnp.float32),
        input_output_aliases={5: 0},
    )(x, W1, b1, W2, b2, prev)


def kernel(batch_inputs, batch_lengths, table, W1, b1, W2, b2):
    del batch_lengths
    ids = batch_inputs.reshape(B * L // IDX_COLS, IDX_COLS)
    rows_per_half = ids.shape[0] // NHALF
    b1r = b1.reshape(1, HID)
    b2r = b2.reshape(1, NCLS)
    out = jnp.zeros((B, NCLS), jnp.float32)
    for h in range(NHALF):
        ids_h = lax.slice_in_dim(ids, h * rows_per_half, (h + 1) * rows_per_half)
        x_h = _gather_sum_half(ids_h, table)
        out = _mlp_into(x_h, W1, b1r, W2, b2r, out,
                        h * (B // NHALF) // 2048)
    return out
